# baseline (speedup 1.0000x reference)
import functools

import jax
import jax.numpy as jnp
from jax import lax
from jax.experimental import pallas as pl
from jax.experimental.pallas import tpu as pltpu
from jax.experimental.pallas import tpu_sc as plsc

B = 4096
K = 50
KP = 64
NW = 32
RPW = B // NW
NC = 2


def _sc_body(tidx_hbm, scores_hbm, logits_hbm, w_out, g_out,
             tidx_v, scor_v, idx_v, w_v, gath_v, tag_v, sem):
    wid = lax.axis_index("s") * NC + lax.axis_index("c")
    rbase = wid * RPW

    pltpu.sync_copy(tidx_hbm.at[pl.ds(rbase * K, RPW * K)],
                    tidx_v.at[pl.ds(0, RPW * K)])
    pltpu.sync_copy(scores_hbm.at[pl.ds(rbase * K, RPW * K)],
                    scor_v.at[pl.ds(0, RPW * K)])

    iota = lax.iota(jnp.int32, 16)
    neg1 = jnp.full((16,), -1, jnp.int32)
    zero16 = jnp.zeros((16,), jnp.float32)

    def init_body(i, _):
        tag_v[pl.ds(i * 16, 16)] = neg1
        return 0

    lax.fori_loop(0, B // 16, init_body, 0)

    def pair_body(t, _):
        for u in range(2):
            i = t * 2 + u
            r = rbase + i
            cs, tags, valids = [], [], []
            for j in range(4):
                c = tidx_v[pl.ds(i * K + j * 16, 16)]
                valid = (c < B) & (c != r)
                if j == 3:
                    valid = valid & (iota < K - 48)
                tag = r * KP + j * 16 + iota
                cc = jnp.minimum(c, B - 1)
                plsc.store_scatter(tag_v, [cc], tag, mask=valid)
                cs.append((c, cc))
                tags.append(tag)
                valids.append(valid)
            for j in range(4):
                c, cc = cs[j]
                valid = valids[j]
                tg = plsc.load_gather(tag_v, [cc], mask=valid)
                win = valid & (tg == tags[j])
                csafe = jnp.where(win, c, r)
                sc_vec = scor_v[pl.ds(i * K + j * 16, 16)]
                idx_v[pl.ds(i * KP + j * 16, 16)] = r * B + csafe
                w_v[pl.ds(i * KP + j * 16, 16)] = jnp.where(win, sc_vec, zero16)
        pltpu.async_copy(
            logits_hbm.at[idx_v.at[pl.ds(t * 2 * KP, 2 * KP)]],
            gath_v.at[pl.ds(t * 2 * KP, 2 * KP)], sem)
        return 0

    lax.fori_loop(0, RPW // 2, pair_body, 0)

    pltpu.make_async_copy(
        w_out.at[pl.ds(0, RPW * KP)], gath_v, sem).wait()


    pltpu.sync_copy(gath_v, g_out.at[pl.ds(rbase * KP, RPW * KP)])
    pltpu.sync_copy(w_v, w_out.at[pl.ds(rbase * KP, RPW * KP)])


def _lse_body(logits_ref, out_ref):
    x = logits_ref[...]
    m = jnp.max(x, axis=1, keepdims=True)
    lse = m + jnp.log(jnp.sum(jnp.exp(x - m), axis=1, keepdims=True))
    part = jnp.sum(lse).reshape(1, 1)

    @pl.when(pl.program_id(0) == 0)
    def _():
        out_ref[...] = jnp.zeros((1, 1), jnp.float32)

    out_ref[...] += part


def _combine_body(w_ref, g_ref, lsesum_ref, out_ref):
    w2 = w_ref[...]
    g2 = g_ref[...]
    acc = jnp.zeros((B // 2, 1), jnp.float32)
    for h in range(2):
        w = w2[:, h * KP:(h + 1) * KP]
        g = g2[:, h * KP:(h + 1) * KP]
        diag = g[:, K:K + 1]
        s_sum = 1.0 + jnp.sum(w, axis=1, keepdims=True)
        dot = diag + jnp.sum(w * g, axis=1, keepdims=True)
        wlogw = jnp.sum(
            jnp.where(w > 0, w * jnp.log(jnp.maximum(w, 1e-30)), 0.0),
            axis=1, keepdims=True)
        acc = acc + (wlogw - dot) / s_sum - jnp.log(s_sum)
    total = jnp.sum(acc).reshape(1, 1) + lsesum_ref[...]
    out_ref[...] = total * (1.0 / B)


def kernel(student_logits, batch_indices, teacher_indices, teacher_scores):
    del batch_indices

    tidx_flat = teacher_indices.reshape(-1)
    scores_flat = teacher_scores.reshape(-1)
    logits_flat = student_logits.reshape(-1)

    sc = functools.partial(
        pl.kernel,
        out_type=[
            jax.ShapeDtypeStruct((B * KP,), jnp.float32),
            jax.ShapeDtypeStruct((B * KP,), jnp.float32),
        ],
        mesh=plsc.VectorSubcoreMesh(core_axis_name="c", subcore_axis_name="s"),
        compiler_params=pltpu.CompilerParams(needs_layout_passes=False),
        scratch_types=[
            pltpu.VMEM((RPW * K + 16,), jnp.int32),
            pltpu.VMEM((RPW * K + 16,), jnp.float32),
            pltpu.VMEM((RPW * KP,), jnp.int32),
            pltpu.VMEM((RPW * KP,), jnp.float32),
            pltpu.VMEM((RPW * KP,), jnp.float32),
            pltpu.VMEM((B,), jnp.int32),
            pltpu.SemaphoreType.DMA,
        ],
    )(_sc_body)

    br = 512
    lsesum = pl.pallas_call(
        _lse_body,
        grid=(B // br,),
        in_specs=[pl.BlockSpec((br, B), lambda i: (i, 0))],
        out_specs=pl.BlockSpec((1, 1), lambda i: (0, 0)),
        out_shape=jax.ShapeDtypeStruct((1, 1), jnp.float32),
    )(student_logits)

    tidx_flat, _ = lax.optimization_barrier((tidx_flat, lsesum))
    w_flat, g_flat = sc(tidx_flat, scores_flat, logits_flat)

    w2 = w_flat.reshape(B // 2, 2 * KP)
    g2 = g_flat.reshape(B // 2, 2 * KP)
    out = pl.pallas_call(
        _combine_body,
        in_specs=[
            pl.BlockSpec((B // 2, 2 * KP), lambda: (0, 0)),
            pl.BlockSpec((B // 2, 2 * KP), lambda: (0, 0)),
            pl.BlockSpec((1, 1), lambda: (0, 0)),
        ],
        out_specs=pl.BlockSpec((1, 1), lambda: (0, 0)),
        out_shape=jax.ShapeDtypeStruct((1, 1), jnp.float32),
    )(w2, g2, lsesum)
    return out[0, 0]


# speedup vs baseline: 1.3937x; 1.3937x over previous
import functools

import jax
import jax.numpy as jnp
from jax import lax
from jax.experimental import pallas as pl
from jax.experimental.pallas import tpu as pltpu
from jax.experimental.pallas import tpu_sc as plsc

B = 4096
K = 50
KP = 64
NW = 32
RPW = B // NW
CR = 8
NCH = RPW // CR
NC = 2


def _sc_body(tidx_hbm, scores_hbm, logits_hbm, w_out, g_out,
             tidx_v, scor_v, w_v, gath_v, tag_v, chunk0, chunk1, s0, s1):
    wid = lax.axis_index("s") * NC + lax.axis_index("c")
    rbase = wid * RPW

    pltpu.sync_copy(tidx_hbm.at[pl.ds(rbase * K, RPW * K)],
                    tidx_v.at[pl.ds(0, RPW * K)])
    pltpu.sync_copy(scores_hbm.at[pl.ds(rbase * K, RPW * K)],
                    scor_v.at[pl.ds(0, RPW * K)])

    iota = lax.iota(jnp.int32, 16)
    neg1 = jnp.full((16,), -1, jnp.int32)
    zero16 = jnp.zeros((16,), jnp.float32)

    def init_body(i, _):
        tag_v[pl.ds(i * 16, 16)] = neg1
        return 0

    lax.fori_loop(0, B // 16, init_body, 0)

    def process(buf, ch):
        def row_body(u, _):
            i = ch * CR + u
            r = rbase + i
            cs, tags, valids = [], [], []
            for j in range(4):
                c = tidx_v[pl.ds(i * K + j * 16, 16)]
                valid = (c < B) & (c != r)
                if j == 3:
                    valid = valid & (iota < K - 48)
                tag = r * KP + j * 16 + iota
                cc = jnp.minimum(c, B - 1)
                plsc.store_scatter(tag_v, [cc], tag, mask=valid)
                cs.append((c, cc))
                tags.append(tag)
                valids.append(valid)
            usplat = jnp.broadcast_to(u, (16,)).astype(jnp.int32)
            for j in range(4):
                c, cc = cs[j]
                valid = valids[j]
                tg = plsc.load_gather(tag_v, [cc], mask=valid)
                win = valid & (tg == tags[j])
                csafe = jnp.where(win, c, r)
                sc_vec = scor_v[pl.ds(i * K + j * 16, 16)]
                w_v[pl.ds(i * KP + j * 16, 16)] = jnp.where(win, sc_vec, zero16)
                gath_v[pl.ds(i * KP + j * 16, 16)] = plsc.load_gather(
                    buf, [usplat, csafe])
            return 0

        lax.fori_loop(0, CR, row_body, 0)

    pltpu.async_copy(logits_hbm.at[pl.ds(rbase, CR)], chunk0, s0)

    def pair_body(t, _):
        pltpu.async_copy(
            logits_hbm.at[pl.ds(rbase + (2 * t + 1) * CR, CR)], chunk1, s1)
        pltpu.make_async_copy(
            logits_hbm.at[pl.ds(0, CR)], chunk0, s0).wait()
        process(chunk0, 2 * t)

        @pl.when(t < NCH // 2 - 1)
        def _():
            pltpu.async_copy(
                logits_hbm.at[pl.ds(rbase + (2 * t + 2) * CR, CR)], chunk0, s0)

        pltpu.make_async_copy(
            logits_hbm.at[pl.ds(0, CR)], chunk1, s1).wait()
        process(chunk1, 2 * t + 1)
        return 0

    lax.fori_loop(0, NCH // 2, pair_body, 0)

    pltpu.sync_copy(gath_v, g_out.at[pl.ds(rbase * KP, RPW * KP)])
    pltpu.sync_copy(w_v, w_out.at[pl.ds(rbase * KP, RPW * KP)])


def _lse_body(logits_ref, out_ref):
    x = logits_ref[...]
    m = jnp.max(x, axis=1, keepdims=True)
    lse = m + jnp.log(jnp.sum(jnp.exp(x - m), axis=1, keepdims=True))
    part = jnp.sum(lse).reshape(1, 1)

    @pl.when(pl.program_id(0) == 0)
    def _():
        out_ref[...] = jnp.zeros((1, 1), jnp.float32)

    out_ref[...] += part


def _combine_body(w_ref, g_ref, lsesum_ref, out_ref):
    w2 = w_ref[...]
    g2 = g_ref[...]
    acc = jnp.zeros((B // 2, 1), jnp.float32)
    for h in range(2):
        w = w2[:, h * KP:(h + 1) * KP]
        g = g2[:, h * KP:(h + 1) * KP]
        diag = g[:, K:K + 1]
        s_sum = 1.0 + jnp.sum(w, axis=1, keepdims=True)
        dot = diag + jnp.sum(w * g, axis=1, keepdims=True)
        wlogw = jnp.sum(
            jnp.where(w > 0, w * jnp.log(jnp.maximum(w, 1e-30)), 0.0),
            axis=1, keepdims=True)
        acc = acc + (wlogw - dot) / s_sum - jnp.log(s_sum)
    total = jnp.sum(acc).reshape(1, 1) + lsesum_ref[...]
    out_ref[...] = total * (1.0 / B)


def kernel(student_logits, batch_indices, teacher_indices, teacher_scores):
    del batch_indices

    tidx_flat = teacher_indices.reshape(-1)
    scores_flat = teacher_scores.reshape(-1)

    sc = functools.partial(
        pl.kernel,
        out_type=[
            jax.ShapeDtypeStruct((B * KP,), jnp.float32),
            jax.ShapeDtypeStruct((B * KP,), jnp.float32),
        ],
        mesh=plsc.VectorSubcoreMesh(core_axis_name="c", subcore_axis_name="s"),
        compiler_params=pltpu.CompilerParams(needs_layout_passes=False),
        scratch_types=[
            pltpu.VMEM((RPW * K + 16,), jnp.int32),
            pltpu.VMEM((RPW * K + 16,), jnp.float32),
            pltpu.VMEM((RPW * KP,), jnp.float32),
            pltpu.VMEM((RPW * KP,), jnp.float32),
            pltpu.VMEM((B,), jnp.int32),
            pltpu.VMEM((CR, B), jnp.float32),
            pltpu.VMEM((CR, B), jnp.float32),
            pltpu.SemaphoreType.DMA,
            pltpu.SemaphoreType.DMA,
        ],
    )(_sc_body)

    br = 512
    lsesum = pl.pallas_call(
        _lse_body,
        grid=(B // br,),
        in_specs=[pl.BlockSpec((br, B), lambda i: (i, 0))],
        out_specs=pl.BlockSpec((1, 1), lambda i: (0, 0)),
        out_shape=jax.ShapeDtypeStruct((1, 1), jnp.float32),
    )(student_logits)

    w_flat, g_flat = sc(tidx_flat, scores_flat, student_logits)

    w2 = w_flat.reshape(B // 2, 2 * KP)
    g2 = g_flat.reshape(B // 2, 2 * KP)
    out = pl.pallas_call(
        _combine_body,
        in_specs=[
            pl.BlockSpec((B // 2, 2 * KP), lambda: (0, 0)),
            pl.BlockSpec((B // 2, 2 * KP), lambda: (0, 0)),
            pl.BlockSpec((1, 1), lambda: (0, 0)),
        ],
        out_specs=pl.BlockSpec((1, 1), lambda: (0, 0)),
        out_shape=jax.ShapeDtypeStruct((1, 1), jnp.float32),
    )(w2, g2, lsesum)
    return out[0, 0]


# speedup vs baseline: 1.4223x; 1.0205x over previous
import functools

import jax
import jax.numpy as jnp
from jax import lax
from jax.experimental import pallas as pl
from jax.experimental.pallas import tpu as pltpu
from jax.experimental.pallas import tpu_sc as plsc

B = 4096
K = 50
KP = 64
NW = 32
RPW = B // NW
CR = 4
NCH = RPW // CR
NC = 2


def _sc_body(tidx_hbm, scores_hbm, logits_hbm, w_out, g_out,
             tidx_v, scor_v, w_v, gath_v, tag_v,
             chunk0, chunk1, chunk2, chunk3, s0, s1, s2, s3):
    bufs = (chunk0, chunk1, chunk2, chunk3)
    sems = (s0, s1, s2, s3)
    wid = lax.axis_index("s") * NC + lax.axis_index("c")
    rbase = wid * RPW

    pltpu.sync_copy(tidx_hbm.at[pl.ds(rbase * K, RPW * K)],
                    tidx_v.at[pl.ds(0, RPW * K)])
    pltpu.sync_copy(scores_hbm.at[pl.ds(rbase * K, RPW * K)],
                    scor_v.at[pl.ds(0, RPW * K)])

    iota = lax.iota(jnp.int32, 16)
    neg1 = jnp.full((16,), -1, jnp.int32)
    zero16 = jnp.zeros((16,), jnp.float32)

    def init_body(i, _):
        tag_v[pl.ds(i * 16, 16)] = neg1
        return 0

    lax.fori_loop(0, B // 16, init_body, 0)

    def process(buf, ch):
        def row_body(u, _):
            i = ch * CR + u
            r = rbase + i
            cs, tags, valids = [], [], []
            for j in range(4):
                c = tidx_v[pl.ds(i * K + j * 16, 16)]
                valid = (c < B) & (c != r)
                if j == 3:
                    valid = valid & (iota < K - 48)
                tag = r * KP + j * 16 + iota
                cc = jnp.minimum(c, B - 1)
                plsc.store_scatter(tag_v, [cc], tag, mask=valid)
                cs.append((c, cc))
                tags.append(tag)
                valids.append(valid)
            usplat = jnp.broadcast_to(u, (16,)).astype(jnp.int32)
            for j in range(4):
                c, cc = cs[j]
                valid = valids[j]
                tg = plsc.load_gather(tag_v, [cc], mask=valid)
                win = valid & (tg == tags[j])
                csafe = jnp.where(win, c, r)
                sc_vec = scor_v[pl.ds(i * K + j * 16, 16)]
                w_v[pl.ds(i * KP + j * 16, 16)] = jnp.where(win, sc_vec, zero16)
                gath_v[pl.ds(i * KP + j * 16, 16)] = plsc.load_gather(
                    buf, [usplat, csafe])
            return 0

        lax.fori_loop(0, CR, row_body, 0)

    for b in range(3):
        pltpu.async_copy(logits_hbm.at[pl.ds(rbase + b * CR, CR)],
                         bufs[b], sems[b])

    def grp_body(t, _):
        for b in range(4):
            ch = 4 * t + b
            nxt = ch + 3

            @pl.when(nxt < NCH)
            def _():
                pltpu.async_copy(
                    logits_hbm.at[pl.ds(rbase + nxt * CR, CR)],
                    bufs[(b + 3) % 4], sems[(b + 3) % 4])

            pltpu.make_async_copy(
                logits_hbm.at[pl.ds(0, CR)], bufs[b], sems[b]).wait()
            process(bufs[b], ch)
        return 0

    lax.fori_loop(0, NCH // 4, grp_body, 0)

    pltpu.sync_copy(gath_v, g_out.at[pl.ds(rbase * KP, RPW * KP)])
    pltpu.sync_copy(w_v, w_out.at[pl.ds(rbase * KP, RPW * KP)])


def _lse_body(logits_ref, out_ref):
    x = logits_ref[...]
    m = jnp.max(x, axis=1, keepdims=True)
    lse = m + jnp.log(jnp.sum(jnp.exp(x - m), axis=1, keepdims=True))
    part = jnp.sum(lse).reshape(1, 1)

    @pl.when(pl.program_id(0) == 0)
    def _():
        out_ref[...] = jnp.zeros((1, 1), jnp.float32)

    out_ref[...] += part


def _combine_body(w_ref, g_ref, lsesum_ref, out_ref):
    w2 = w_ref[...]
    g2 = g_ref[...]
    acc = jnp.zeros((B // 2, 1), jnp.float32)
    for h in range(2):
        w = w2[:, h * KP:(h + 1) * KP]
        g = g2[:, h * KP:(h + 1) * KP]
        diag = g[:, K:K + 1]
        s_sum = 1.0 + jnp.sum(w, axis=1, keepdims=True)
        dot = diag + jnp.sum(w * g, axis=1, keepdims=True)
        wlogw = jnp.sum(
            jnp.where(w > 0, w * jnp.log(jnp.maximum(w, 1e-30)), 0.0),
            axis=1, keepdims=True)
        acc = acc + (wlogw - dot) / s_sum - jnp.log(s_sum)
    total = jnp.sum(acc).reshape(1, 1) + lsesum_ref[...]
    out_ref[...] = total * (1.0 / B)


def kernel(student_logits, batch_indices, teacher_indices, teacher_scores):
    del batch_indices

    tidx_flat = teacher_indices.reshape(-1)
    scores_flat = teacher_scores.reshape(-1)

    sc = functools.partial(
        pl.kernel,
        out_type=[
            jax.ShapeDtypeStruct((B * KP,), jnp.float32),
            jax.ShapeDtypeStruct((B * KP,), jnp.float32),
        ],
        mesh=plsc.VectorSubcoreMesh(core_axis_name="c", subcore_axis_name="s"),
        compiler_params=pltpu.CompilerParams(needs_layout_passes=False),
        scratch_types=[
            pltpu.VMEM((RPW * K + 16,), jnp.int32),
            pltpu.VMEM((RPW * K + 16,), jnp.float32),
            pltpu.VMEM((RPW * KP,), jnp.float32),
            pltpu.VMEM((RPW * KP,), jnp.float32),
            pltpu.VMEM((B,), jnp.int32),
            pltpu.VMEM((CR, B), jnp.float32),
            pltpu.VMEM((CR, B), jnp.float32),
            pltpu.VMEM((CR, B), jnp.float32),
            pltpu.VMEM((CR, B), jnp.float32),
            pltpu.SemaphoreType.DMA,
            pltpu.SemaphoreType.DMA,
            pltpu.SemaphoreType.DMA,
            pltpu.SemaphoreType.DMA,
        ],
    )(_sc_body)

    br = 512
    lsesum = pl.pallas_call(
        _lse_body,
        grid=(B // br,),
        in_specs=[pl.BlockSpec((br, B), lambda i: (i, 0))],
        out_specs=pl.BlockSpec((1, 1), lambda i: (0, 0)),
        out_shape=jax.ShapeDtypeStruct((1, 1), jnp.float32),
    )(student_logits)

    w_flat, g_flat = sc(tidx_flat, scores_flat, student_logits)

    w2 = w_flat.reshape(B // 2, 2 * KP)
    g2 = g_flat.reshape(B // 2, 2 * KP)
    out = pl.pallas_call(
        _combine_body,
        in_specs=[
            pl.BlockSpec((B // 2, 2 * KP), lambda: (0, 0)),
            pl.BlockSpec((B // 2, 2 * KP), lambda: (0, 0)),
            pl.BlockSpec((1, 1), lambda: (0, 0)),
        ],
        out_specs=pl.BlockSpec((1, 1), lambda: (0, 0)),
        out_shape=jax.ShapeDtypeStruct((1, 1), jnp.float32),
    )(w2, g2, lsesum)
    return out[0, 0]
